# baseline (speedup 1.0000x reference)
import jax
import jax.numpy as jnp
from jax.experimental import pallas as pl
from jax.experimental.pallas import tpu as pltpu

D = 2048
N_TOK = 32768
B = 16
R = 2048
NR = 2
RS = R // NR
NB = N_TOK // R
CW = 1024
NJ = D // CW
NEG = -1e30


def _body(sidx_ref, idx_ref, watt_ref, batt_ref, wout_ref, bout_ref, *refs):
    a_refs = refs[:NR]
    out_ref, m_ref, l_ref, acc_ref = refs[NR:NR + 4]
    i = pl.program_id(0)
    b_lo = sidx_ref[0]
    b_hi = sidx_ref[1]

    @pl.when(i == 0)
    def _init():
        m_ref[...] = jnp.full_like(m_ref, NEG)
        l_ref[...] = jnp.zeros_like(l_ref)
        acc_ref[...] = jnp.zeros_like(acc_ref)

    @pl.when(b_lo + i * NR <= b_hi)
    def _pool():
        a = [r[...] for r in a_refs]
        w = watt_ref[...]
        st = idx_ref[...][:, 0][None, :]
        en = idx_ref[...][:, 1][None, :]

        sbs = []
        for k in range(NR):
            s = jax.lax.dot_general(
                a[k], w, (((1,), (0,)), ((), ())),
                preferred_element_type=jnp.float32) + batt_ref[0, 0]
            pos = (b_lo + i * NR + k) * RS + jax.lax.broadcasted_iota(
                jnp.int32, (RS, B), 0)
            mask = (pos >= st) & (pos <= en)
            sbs.append(jnp.where(mask, s, NEG))

        bm = sbs[0].max(axis=0)
        for k in range(1, NR):
            bm = jnp.maximum(bm, sbs[k].max(axis=0))
        m_old = m_ref[0]
        m_new = jnp.maximum(m_old, bm)
        alpha = jnp.exp(m_old - m_new)
        es = [jnp.exp(sb - m_new[None, :]) for sb in sbs]
        lsum = es[0].sum(axis=0)
        for k in range(1, NR):
            lsum = lsum + es[k].sum(axis=0)
        l_ref[0] = alpha * l_ref[0] + lsum
        m_ref[0] = m_new
        upd = jax.lax.dot_general(es[0], a[0], (((0,), (0,)), ((), ())),
                                  preferred_element_type=jnp.float32)
        for k in range(1, NR):
            upd = upd + jax.lax.dot_general(
                es[k], a[k], (((0,), (0,)), ((), ())),
                preferred_element_type=jnp.float32)
        acc_ref[...] = acc_ref[...] * alpha[:, None] + upd

    @pl.when(i >= NB)
    def _project():
        pooled = acc_ref[...] / l_ref[0][:, None]
        out_ref[...] = jax.lax.dot_general(
            pooled, wout_ref[...], (((1,), (0,)), ((), ())),
            preferred_element_type=jnp.float32) + bout_ref[...]


def _a_spec(k):
    def imap(i, sidx_ref):
        v = sidx_ref[0] + i * NR + k
        return (jnp.minimum(v, sidx_ref[1]), 0)
    return pl.BlockSpec((RS, D), imap)


def _jmap(i, sidx_ref):
    del sidx_ref
    return (0, jnp.maximum(i - NB, 0))


@jax.jit
def kernel(atom_features, index_list, W_att, b_att, W_out, b_out):
    idx32 = index_list.astype(jnp.int32)
    limits = jnp.stack([jnp.min(idx32[:, 0]) // RS,
                        jnp.max(idx32[:, 1]) // RS])
    return pl.pallas_call(
        _body,
        grid_spec=pltpu.PrefetchScalarGridSpec(
            num_scalar_prefetch=1,
            grid=(NB + NJ,),
            in_specs=[
                pl.BlockSpec((B, 2), lambda i, s: (0, 0)),
                pl.BlockSpec((D, 1), lambda i, s: (0, 0)),
                pl.BlockSpec((1, 1), lambda i, s: (0, 0)),
                pl.BlockSpec((D, CW), _jmap),
                pl.BlockSpec((1, CW), _jmap),
            ] + [_a_spec(k) for k in range(NR)],
            out_specs=pl.BlockSpec((B, CW), _jmap),
            scratch_shapes=[
                pltpu.VMEM((1, B), jnp.float32),
                pltpu.VMEM((1, B), jnp.float32),
                pltpu.VMEM((B, D), jnp.float32),
            ],
        ),
        out_shape=jax.ShapeDtypeStruct((B, D), jnp.float32),
    )(limits, idx32, W_att, b_att.reshape(1, 1), W_out, b_out.reshape(1, D),
      *([atom_features] * NR))


# speedup vs baseline: 1.0960x; 1.0960x over previous
import jax
import jax.numpy as jnp
from jax.experimental import pallas as pl
from jax.experimental.pallas import tpu as pltpu

D = 2048
N_TOK = 32768
B = 16
R = 2048
NR = 2
RS = R // NR
NB = N_TOK // R
CW = 1024
NJ = D // CW
NEG = -1e30


def _first_sub(idx_ref):
    m = idx_ref[0, 0]
    for b in range(1, B):
        m = jnp.minimum(m, idx_ref[b, 0])
    return m // RS


def _last_sub(idx_ref):
    m = idx_ref[0, 1]
    for b in range(1, B):
        m = jnp.maximum(m, idx_ref[b, 1])
    return m // RS


def _body(sidx_ref, idx_ref, watt_ref, batt_ref, wout_ref, bout_ref, *refs):
    a_refs = refs[:NR]
    out_ref, m_ref, l_ref, acc_ref = refs[NR:NR + 4]
    i = pl.program_id(0)
    b_lo = _first_sub(sidx_ref)
    b_hi = _last_sub(sidx_ref)

    @pl.when(i == 0)
    def _init():
        m_ref[...] = jnp.full_like(m_ref, NEG)
        l_ref[...] = jnp.zeros_like(l_ref)
        acc_ref[...] = jnp.zeros_like(acc_ref)

    @pl.when(b_lo + i * NR <= b_hi)
    def _pool():
        a = [r[...] for r in a_refs]
        w = watt_ref[...]
        st = idx_ref[...][:, 0][None, :]
        en = idx_ref[...][:, 1][None, :]

        sbs = []
        for k in range(NR):
            s = jax.lax.dot_general(
                a[k], w, (((1,), (0,)), ((), ())),
                preferred_element_type=jnp.float32) + batt_ref[0, 0]
            pos = (b_lo + i * NR + k) * RS + jax.lax.broadcasted_iota(
                jnp.int32, (RS, B), 0)
            mask = (pos >= st) & (pos <= en)
            sbs.append(jnp.where(mask, s, NEG))

        bm = sbs[0].max(axis=0)
        for k in range(1, NR):
            bm = jnp.maximum(bm, sbs[k].max(axis=0))
        m_old = m_ref[0]
        m_new = jnp.maximum(m_old, bm)
        alpha = jnp.exp(m_old - m_new)
        es = [jnp.exp(sb - m_new[None, :]) for sb in sbs]
        lsum = es[0].sum(axis=0)
        for k in range(1, NR):
            lsum = lsum + es[k].sum(axis=0)
        l_ref[0] = alpha * l_ref[0] + lsum
        m_ref[0] = m_new
        upd = jax.lax.dot_general(es[0], a[0], (((0,), (0,)), ((), ())),
                                  preferred_element_type=jnp.float32)
        for k in range(1, NR):
            upd = upd + jax.lax.dot_general(
                es[k], a[k], (((0,), (0,)), ((), ())),
                preferred_element_type=jnp.float32)
        acc_ref[...] = acc_ref[...] * alpha[:, None] + upd

    @pl.when(i >= NB)
    def _project():
        pooled = acc_ref[...] / l_ref[0][:, None]
        out_ref[...] = jax.lax.dot_general(
            pooled, wout_ref[...], (((1,), (0,)), ((), ())),
            preferred_element_type=jnp.float32) + bout_ref[...]


def _a_spec(k):
    def imap(i, sidx_ref):
        v = _first_sub(sidx_ref) + i * NR + k
        return (jnp.minimum(v, _last_sub(sidx_ref)), 0)
    return pl.BlockSpec((RS, D), imap)


def _jmap(i, sidx_ref):
    del sidx_ref
    return (0, jnp.maximum(i - NB, 0))


@jax.jit
def kernel(atom_features, index_list, W_att, b_att, W_out, b_out):
    idx32 = index_list.astype(jnp.int32)
    return pl.pallas_call(
        _body,
        grid_spec=pltpu.PrefetchScalarGridSpec(
            num_scalar_prefetch=1,
            grid=(NB + NJ,),
            in_specs=[
                pl.BlockSpec((B, 2), lambda i, s: (0, 0)),
                pl.BlockSpec((D, 1), lambda i, s: (0, 0)),
                pl.BlockSpec((1, 1), lambda i, s: (0, 0)),
                pl.BlockSpec((D, CW), _jmap),
                pl.BlockSpec((1, CW), _jmap),
            ] + [_a_spec(k) for k in range(NR)],
            out_specs=pl.BlockSpec((B, CW), _jmap),
            scratch_shapes=[
                pltpu.VMEM((1, B), jnp.float32),
                pltpu.VMEM((1, B), jnp.float32),
                pltpu.VMEM((B, D), jnp.float32),
            ],
        ),
        out_shape=jax.ShapeDtypeStruct((B, D), jnp.float32),
    )(idx32, idx32, W_att, b_att.reshape(1, 1), W_out, b_out.reshape(1, D),
      *([atom_features] * NR))
